# Initial kernel scaffold; baseline (speedup 1.0000x reference)
import functools
import jax
import jax.numpy as jnp
from jax import lax
from jax.experimental import pallas as pl
from jax.experimental.pallas import tpu as pltpu
from jax.experimental.pallas import tpu_sc as plsc

_N = 10000
_D = 256
_H = 256
_C = 64
_E = 160000
_L = 2

_NC = 2
_NS = 16
_HH = _H // 2
_EPT = _E // _NS
_CH = 125
_NCHUNK = _EPT // _CH
_RPT = _N // _NS
_NZ = _RPT // _CH

_SRC_T = (0, 1, 1, 2)
_DST_T = (1, 0, 2, 1)

_BN = 1000
_NB = _N // _BN


def _sc_segment_sums(h_stack, src_all, dst_all, with_cnt):
    mesh = plsc.VectorSubcoreMesh(core_axis_name="c", subcore_axis_name="s")
    out_type = [jax.ShapeDtypeStruct((4, _NC, _N, _HH), jnp.float32)]
    if with_cnt:
        out_type.append(jax.ShapeDtypeStruct((4, _N, 16), jnp.float32))

    def body(h_ref, src_ref, dst_ref, *rest):
        if with_cnt:
            out_ref, cnt_ref = rest[0], rest[1]
            scratch = rest[2:]
        else:
            out_ref = rest[0]
            cnt_ref = None
            scratch = rest[1:]
        (src_v, dst_v, rows_v, ones_v, zrow_v, zcnt_v,
         acc_sh, cnt_sh, sem) = scratch
        c = lax.axis_index("c")
        s = lax.axis_index("s")
        row0 = s * _RPT

        zeros16 = jnp.zeros((16,), jnp.float32)
        ones16 = jnp.ones((16,), jnp.float32)

        @pl.loop(0, _CH)
        def _fill_ones(i):
            ones_v[i] = ones16
            for jj in range(_HH // 16):
                zrow_v[i, pl.ds(jj * 16, 16)] = zeros16

        @pl.loop(0, _RPT)
        def _fill_zcnt(i):
            zcnt_v[i] = zeros16

        for r in range(4):
            tab = h_ref.at[_SRC_T[r]]
            for z in range(_NZ):
                pltpu.sync_copy(zrow_v, acc_sh.at[pl.ds(row0 + z * _CH, _CH)])
            if with_cnt:
                pltpu.sync_copy(zcnt_v, cnt_sh.at[pl.ds(row0, _RPT)])
            pltpu.sync_copy(src_ref.at[r, s], src_v)
            pltpu.sync_copy(dst_ref.at[r, s], dst_v)
            plsc.subcore_barrier()

            @pl.loop(0, _NCHUNK)
            def _chunk(j):
                srow = src_v.at[j]
                drow = dst_v.at[j]
                pltpu.async_copy(tab.at[c].at[srow], rows_v, sem).wait()
                pltpu.sync_copy(rows_v, acc_sh.at[drow], add=True)
                if with_cnt:
                    pltpu.sync_copy(ones_v, cnt_sh.at[drow], add=True)

            plsc.subcore_barrier()
            pltpu.sync_copy(acc_sh.at[pl.ds(row0, _RPT)],
                            out_ref.at[r, c, pl.ds(row0, _RPT)])
            if with_cnt:
                @pl.when(c == 0)
                def _():
                    pltpu.sync_copy(cnt_sh.at[pl.ds(row0, _RPT)],
                                    cnt_ref.at[r, pl.ds(row0, _RPT)])

    k = pl.kernel(
        body,
        out_type=tuple(out_type),
        mesh=mesh,
        scratch_types=[
            pltpu.VMEM((_NCHUNK, _CH), jnp.int32),
            pltpu.VMEM((_NCHUNK, _CH), jnp.int32),
            pltpu.VMEM((_CH, _HH), jnp.float32),
            pltpu.VMEM((_CH, 16), jnp.float32),
            pltpu.VMEM((_CH, _HH), jnp.float32),
            pltpu.VMEM((_RPT, 16), jnp.float32),
            pltpu.VMEM_SHARED((_N, _HH), jnp.float32),
            pltpu.VMEM_SHARED((_N, 16), jnp.float32),
            pltpu.SemaphoreType.DMA,
        ],
        name="sc_segsum" + ("_cnt" if with_cnt else ""),
    )
    return k(h_stack, src_all, dst_all)


def _tc_project(xs, Wps, bps):
    def body(x_ref, w_ref, b_ref, o_ref):
        o_ref[0, 0] = (
            jnp.dot(x_ref[0], w_ref[0], preferred_element_type=jnp.float32)
            + b_ref[0]
        )

    return pl.pallas_call(
        body,
        grid=(3, _NB, _NC),
        in_specs=[
            pl.BlockSpec((1, _BN, _D), lambda t, i, c: (t, i, 0)),
            pl.BlockSpec((1, _D, _HH), lambda t, i, c: (t, 0, c)),
            pl.BlockSpec((1, _HH), lambda t, i, c: (t, c)),
        ],
        out_specs=pl.BlockSpec((1, 1, _BN, _HH), lambda t, i, c: (t, c, i, 0)),
        out_shape=jax.ShapeDtypeStruct((3, _NC, _N, _HH), jnp.float32),
        name="tc_project",
    )(xs, Wps, bps)


def _tc_layer_combine(sums, cnt, h_stack, Wl_l, bl_l, Wr_l):
    def body(s_ref, c_ref, h_ref, wl_ref, bl_ref, wr_ref, o_ref):
        o = []
        for r in range(4):
            acc = jnp.dot(s_ref[r, 0], wl_ref[r, :_HH, :],
                          preferred_element_type=jnp.float32)
            acc = acc + jnp.dot(s_ref[r, 1], wl_ref[r, _HH:, :],
                                preferred_element_type=jnp.float32)
            inv = 1.0 / jnp.maximum(c_ref[r, :, 0:1], 1.0)
            acc = acc * inv + bl_ref[r]
            t = _DST_T[r]
            acc = acc + jnp.dot(h_ref[t, 0], wr_ref[r, :_HH, :],
                                preferred_element_type=jnp.float32)
            acc = acc + jnp.dot(h_ref[t, 1], wr_ref[r, _HH:, :],
                                preferred_element_type=jnp.float32)
            o.append(acc)
        new = (jnp.maximum(o[1], 0.0),
               jnp.maximum((o[0] + o[3]) * 0.5, 0.0),
               jnp.maximum(o[2], 0.0))
        for t in range(3):
            o_ref[t, 0] = new[t][:, :_HH]
            o_ref[t, 1] = new[t][:, _HH:]

    return pl.pallas_call(
        body,
        grid=(_NB,),
        in_specs=[
            pl.BlockSpec((4, _NC, _BN, _HH), lambda i: (0, 0, i, 0)),
            pl.BlockSpec((4, _BN, 16), lambda i: (0, i, 0)),
            pl.BlockSpec((3, _NC, _BN, _HH), lambda i: (0, 0, i, 0)),
            pl.BlockSpec((4, _H, _H), lambda i: (0, 0, 0)),
            pl.BlockSpec((4, _H), lambda i: (0, 0)),
            pl.BlockSpec((4, _H, _H), lambda i: (0, 0, 0)),
        ],
        out_specs=pl.BlockSpec((3, _NC, _BN, _HH), lambda i: (0, 0, i, 0)),
        out_shape=jax.ShapeDtypeStruct((3, _NC, _N, _HH), jnp.float32),
        name="tc_layer_combine",
    )(sums, cnt, h_stack, Wl_l, bl_l, Wr_l)


def _tc_classifier(h_stack, Wc, bc):
    def body(h_ref, w_ref, b_ref, o_ref):
        o_ref[...] = (
            jnp.dot(h_ref[0, 0], w_ref[:_HH, :],
                    preferred_element_type=jnp.float32)
            + jnp.dot(h_ref[0, 1], w_ref[_HH:, :],
                      preferred_element_type=jnp.float32)
            + b_ref[...]
        )

    return pl.pallas_call(
        body,
        grid=(_NB,),
        in_specs=[
            pl.BlockSpec((1, _NC, _BN, _HH), lambda i: (0, 0, i, 0)),
            pl.BlockSpec((_H, _C), lambda i: (0, 0)),
            pl.BlockSpec((1, _C), lambda i: (0, 0)),
        ],
        out_specs=pl.BlockSpec((_BN, _C), lambda i: (i, 0)),
        out_shape=jax.ShapeDtypeStruct((_N, _C), jnp.float32),
        name="tc_classifier",
    )(h_stack, Wc, bc)


def kernel(x_lncRNA, x_miRNA, x_mRNA, edge_index_interacts,
           edge_index_rev_interacts, edge_index_regulates,
           edge_index_rev_regulates, Wp_lncRNA, bp_lncRNA, Wp_miRNA, bp_miRNA,
           Wp_mRNA, bp_mRNA, Wl, bl, Wr, Wc, bc):
    xs = jnp.stack([x_lncRNA, x_miRNA, x_mRNA])
    Wps = jnp.stack([Wp_lncRNA, Wp_miRNA, Wp_mRNA])
    bps = jnp.stack([bp_lncRNA, bp_miRNA, bp_mRNA])

    eis = (edge_index_interacts, edge_index_rev_interacts,
           edge_index_regulates, edge_index_rev_regulates)
    src_all = jnp.stack([e[0].reshape(_NS, _NCHUNK, _CH) for e in eis])
    dst_all = jnp.stack([e[1].reshape(_NS, _NCHUNK, _CH) for e in eis])

    h = _tc_project(xs, Wps, bps)

    cnt = None
    for l in range(_L):
        if l == 0:
            sums, cnt = _sc_segment_sums(h, src_all, dst_all, with_cnt=True)
        else:
            (sums,) = _sc_segment_sums(h, src_all, dst_all, with_cnt=False)
        h = _tc_layer_combine(sums, cnt, h, Wl[l], bl[l], Wr[l])

    return _tc_classifier(h, Wc, bc)


# speedup vs baseline: 1.3862x; 1.3862x over previous
import functools
import jax
import jax.numpy as jnp
from jax import lax
from jax.experimental import pallas as pl
from jax.experimental.pallas import tpu as pltpu
from jax.experimental.pallas import tpu_sc as plsc

_N = 10000
_D = 256
_H = 256
_C = 64
_E = 160000
_L = 2

_NC = 2
_NS = 16
_HH = _H // 2
_EPT = _E // _NS
_CH = 128
_EPP = 10240
_NCHUNK = _EPP // _CH
_NP = 10240
_RPT = _NP // _NS
_ZCH = 64
_NZ = _RPT // _ZCH
_IG = 16
_NG = _NCHUNK // _IG

_SRC_T = (0, 1, 1, 2)
_DST_T = (1, 0, 2, 1)

_BN = 1024
_NB = _NP // _BN


def _sc_segment_sums(h_stack, src_all, dst_all, with_cnt):
    mesh = plsc.VectorSubcoreMesh(core_axis_name="c", subcore_axis_name="s")
    out_type = [jax.ShapeDtypeStruct((4, _NC, _NP, _HH), jnp.float32)]
    if with_cnt:
        out_type.append(jax.ShapeDtypeStruct((_NS * 4 * _NP,), jnp.float32))

    def body(h_ref, src_ref, dst_ref, *rest):
        if with_cnt:
            out_ref, cnt_ref = rest[0], rest[1]
            scratch = rest[2:]
        else:
            out_ref = rest[0]
            cnt_ref = None
            scratch = rest[1:]
        (src_v, dst_v, rows_v, zrow_v, hist_v, acc_sh, sem) = scratch
        c = lax.axis_index("c")
        s = lax.axis_index("s")
        row0 = s * _RPT

        zeros16 = jnp.zeros((16,), jnp.float32)
        ones16 = jnp.ones((16,), jnp.float32)

        @pl.loop(0, _ZCH)
        def _fill_zrow(i):
            for jj in range(_HH // 16):
                zrow_v[i, pl.ds(jj * 16, 16)] = zeros16

        for r in range(4):
            tab = h_ref.at[_SRC_T[r]]
            for z in range(_NZ):
                pltpu.sync_copy(zrow_v, acc_sh.at[pl.ds(row0 + z * _ZCH, _ZCH)])
            if with_cnt:
                @pl.when(c == 0)
                def _hz():
                    @pl.loop(0, _NP // 16)
                    def _hzl(i):
                        hist_v[pl.ds(i * 16, 16)] = zeros16
            plsc.subcore_barrier()

            @pl.loop(0, _NG)
            def _group(g):
                pltpu.sync_copy(src_ref.at[r, s, pl.ds(g * _IG, _IG)], src_v)
                pltpu.sync_copy(dst_ref.at[r, s, pl.ds(g * _IG, _IG)], dst_v)

                @pl.loop(0, _IG)
                def _chunk(j):
                    srow = src_v.at[j]
                    drow = dst_v.at[j]
                    pltpu.async_copy(tab.at[c].at[srow], rows_v, sem).wait()
                    pltpu.sync_copy(rows_v, acc_sh.at[drow], add=True)
                    if with_cnt:
                        @pl.when(c == 0)
                        def _hacc():
                            for gg in range(_CH // 16):
                                idx = dst_v[j, pl.ds(gg * 16, 16)]
                                plsc.addupdate_scatter(hist_v, [idx], ones16)

            if with_cnt:
                @pl.when(c == 0)
                def _hout():
                    pltpu.sync_copy(
                        hist_v,
                        cnt_ref.at[pl.ds((s * 4 + r) * _NP, _NP)])

            plsc.subcore_barrier()
            pltpu.sync_copy(acc_sh.at[pl.ds(row0, _RPT)],
                            out_ref.at[r, c, pl.ds(row0, _RPT)])

    k = pl.kernel(
        body,
        out_type=tuple(out_type),
        mesh=mesh,
        scratch_types=[
            pltpu.VMEM((_IG, _CH), jnp.int32),
            pltpu.VMEM((_IG, _CH), jnp.int32),
            pltpu.VMEM((_CH, _HH), jnp.float32),
            pltpu.VMEM((_ZCH, _HH), jnp.float32),
            pltpu.VMEM((_NP,), jnp.float32),
            pltpu.VMEM_SHARED((_NP, _HH), jnp.float32),
            pltpu.SemaphoreType.DMA,
        ],
        compiler_params=pltpu.CompilerParams(
            needs_layout_passes=False, use_tc_tiling_on_sc=False),
        name="sc_segsum" + ("_cnt" if with_cnt else ""),
    )
    return k(h_stack, src_all, dst_all)


def _tc_project(xs, Wps, bps):
    def body(x_ref, w_ref, b_ref, o_ref):
        o_ref[0, 0] = (
            jnp.dot(x_ref[0], w_ref[0], preferred_element_type=jnp.float32)
            + b_ref[0, 0]
        )

    return pl.pallas_call(
        body,
        grid=(3, _NB, _NC),
        in_specs=[
            pl.BlockSpec((1, _BN, _D), lambda t, i, c: (t, i, 0)),
            pl.BlockSpec((1, _D, _HH), lambda t, i, c: (t, 0, c)),
            pl.BlockSpec((1, 1, 1, _HH), lambda t, i, c: (t, c, 0, 0)),
        ],
        out_specs=pl.BlockSpec((1, 1, _BN, _HH), lambda t, i, c: (t, c, i, 0)),
        out_shape=jax.ShapeDtypeStruct((3, _NC, _NP, _HH), jnp.float32),
        name="tc_project",
    )(xs, Wps, bps)


def _tc_layer_combine(sums, cnt, h_stack, Wl_l, bl_l, Wr_l):
    def body(s_ref, c_ref, h_ref, wl_ref, bl_ref, wr_ref, o_ref):
        cnts = jnp.sum(c_ref[...], axis=0)
        o = []
        for r in range(4):
            acc = jnp.dot(s_ref[r, 0], wl_ref[r, :_HH, :],
                          preferred_element_type=jnp.float32)
            acc = acc + jnp.dot(s_ref[r, 1], wl_ref[r, _HH:, :],
                                preferred_element_type=jnp.float32)
            inv = 1.0 / jnp.maximum(cnts[r][:, None], 1.0)
            acc = acc * inv + bl_ref[r]
            t = _DST_T[r]
            acc = acc + jnp.dot(h_ref[t, 0], wr_ref[r, :_HH, :],
                                preferred_element_type=jnp.float32)
            acc = acc + jnp.dot(h_ref[t, 1], wr_ref[r, _HH:, :],
                                preferred_element_type=jnp.float32)
            o.append(acc)
        new = (jnp.maximum(o[1], 0.0),
               jnp.maximum((o[0] + o[3]) * 0.5, 0.0),
               jnp.maximum(o[2], 0.0))
        for t in range(3):
            o_ref[t, 0] = new[t][:, :_HH]
            o_ref[t, 1] = new[t][:, _HH:]

    return pl.pallas_call(
        body,
        grid=(_NB,),
        in_specs=[
            pl.BlockSpec((4, _NC, _BN, _HH), lambda i: (0, 0, i, 0)),
            pl.BlockSpec((_NS, 4, _BN), lambda i: (0, 0, i)),
            pl.BlockSpec((3, _NC, _BN, _HH), lambda i: (0, 0, i, 0)),
            pl.BlockSpec((4, _H, _H), lambda i: (0, 0, 0)),
            pl.BlockSpec((4, _H), lambda i: (0, 0)),
            pl.BlockSpec((4, _H, _H), lambda i: (0, 0, 0)),
        ],
        out_specs=pl.BlockSpec((3, _NC, _BN, _HH), lambda i: (0, 0, i, 0)),
        out_shape=jax.ShapeDtypeStruct((3, _NC, _NP, _HH), jnp.float32),
        name="tc_layer_combine",
    )(sums, cnt, h_stack, Wl_l, bl_l, Wr_l)


def _tc_classifier(h_stack, Wc, bc):
    def body(h_ref, w_ref, b_ref, o_ref):
        o_ref[...] = (
            jnp.dot(h_ref[0, 0], w_ref[:_HH, :],
                    preferred_element_type=jnp.float32)
            + jnp.dot(h_ref[0, 1], w_ref[_HH:, :],
                      preferred_element_type=jnp.float32)
            + b_ref[...]
        )

    return pl.pallas_call(
        body,
        grid=(_NB,),
        in_specs=[
            pl.BlockSpec((1, _NC, _BN, _HH), lambda i: (0, 0, i, 0)),
            pl.BlockSpec((_H, _C), lambda i: (0, 0)),
            pl.BlockSpec((1, _C), lambda i: (0, 0)),
        ],
        out_specs=pl.BlockSpec((_BN, _C), lambda i: (i, 0)),
        out_shape=jax.ShapeDtypeStruct((_NP, _C), jnp.float32),
        name="tc_classifier",
    )(h_stack, Wc, bc.reshape(1, _C))


def kernel(x_lncRNA, x_miRNA, x_mRNA, edge_index_interacts,
           edge_index_rev_interacts, edge_index_regulates,
           edge_index_rev_regulates, Wp_lncRNA, bp_lncRNA, Wp_miRNA, bp_miRNA,
           Wp_mRNA, bp_mRNA, Wl, bl, Wr, Wc, bc):
    xs = jnp.stack([x_lncRNA, x_miRNA, x_mRNA])
    xs = jnp.pad(xs, ((0, 0), (0, _NP - _N), (0, 0)))
    Wps = jnp.stack([Wp_lncRNA, Wp_miRNA, Wp_mRNA])
    bps = jnp.stack([bp_lncRNA, bp_miRNA, bp_mRNA]).reshape(3, _NC, 1, _HH)

    eis = (edge_index_interacts, edge_index_rev_interacts,
           edge_index_regulates, edge_index_rev_regulates)
    pad_row = jnp.int32(_NP - 1)

    def _prep(row):
        row = row.reshape(_NS, _EPT)
        row = jnp.pad(row, ((0, 0), (0, _EPP - _EPT)), constant_values=pad_row)
        return row.reshape(_NS, _NCHUNK, _CH)

    src_all = jnp.stack([_prep(e[0]) for e in eis])
    dst_all = jnp.stack([_prep(e[1]) for e in eis])

    h = _tc_project(xs, Wps, bps)

    cnt = None
    for l in range(_L):
        if l == 0:
            sums, cnt1d = _sc_segment_sums(h, src_all, dst_all, with_cnt=True)
            cnt = cnt1d.reshape(_NS, 4, _NP)
        else:
            (sums,) = _sc_segment_sums(h, src_all, dst_all, with_cnt=False)
        h = _tc_layer_combine(sums, cnt, h, Wl[l], bl[l], Wr[l])

    return _tc_classifier(h, Wc, bc)[:_N]


# speedup vs baseline: 1.7225x; 1.2426x over previous
import functools
import jax
import jax.numpy as jnp
from jax import lax
from jax.experimental import pallas as pl
from jax.experimental.pallas import tpu as pltpu
from jax.experimental.pallas import tpu_sc as plsc

_N = 10000
_D = 256
_H = 256
_C = 64
_E = 160000
_L = 2

_NC = 2
_NS = 16
_HH = _H // 2
_EPT = _E // _NS
_CH = 128
_EPP = 10240
_NCHUNK = _EPP // _CH
_NP = 10240
_RPT = _NP // _NS
_ZCH = 64
_NZ = _RPT // _ZCH
_IG = 16
_NG = _NCHUNK // _IG

_SRC_T = (0, 1, 1, 2)
_DST_T = (1, 0, 2, 1)

_BN = 1024
_NB = _NP // _BN


def _sc_segment_sums(h_stack, src_all, dst_all, zeros_hbm):
    mesh = plsc.VectorSubcoreMesh(core_axis_name="c", subcore_axis_name="s")

    def body(h_ref, src_ref, dst_ref, z_ref, out_ref,
             sa_v, sb_v, da_v, db_v, rows_a, rows_b,
             isem, gsem_a, gsem_b, ssem_a, ssem_b):
        c = lax.axis_index("c")
        s = lax.axis_index("s")
        row0 = s * _RPT

        for r in range(4):
            tab = h_ref.at[_SRC_T[r]]
            pltpu.sync_copy(z_ref, _ACC[0].at[pl.ds(row0, _RPT)])
            plsc.subcore_barrier()

            bufs = ((sa_v, da_v, rows_a), (sb_v, db_v, rows_b))
            pltpu.sync_copy(src_ref.at[r, s, pl.ds(0, _IG)], sa_v)
            pltpu.sync_copy(dst_ref.at[r, s, pl.ds(0, _IG)], da_v)
            for g in range(_NG):
                s_cur, d_cur, _ = bufs[g % 2]
                s_nxt, d_nxt, _ = bufs[(g + 1) % 2]
                if g + 1 < _NG:
                    pltpu.async_copy(
                        src_ref.at[r, s, pl.ds((g + 1) * _IG, _IG)], s_nxt,
                        isem)
                    pltpu.async_copy(
                        dst_ref.at[r, s, pl.ds((g + 1) * _IG, _IG)], d_nxt,
                        isem)

                pltpu.async_copy(tab.at[c].at[s_cur.at[0]], rows_a, gsem_a)
                pltpu.async_copy(tab.at[c].at[s_cur.at[1]], rows_b, gsem_b)
                pltpu.make_async_copy(tab.at[c].at[s_cur.at[0]], rows_a,
                                      gsem_a).wait()
                pltpu.async_copy(rows_a, _ACC[0].at[d_cur.at[0]], ssem_a,
                                 add=True)
                pltpu.make_async_copy(tab.at[c].at[s_cur.at[1]], rows_b,
                                      gsem_b).wait()
                pltpu.async_copy(rows_b, _ACC[0].at[d_cur.at[1]], ssem_b,
                                 add=True)

                @pl.loop(1, _IG // 2)
                def _pair(p):
                    c0 = 2 * p
                    c1 = 2 * p + 1
                    pltpu.make_async_copy(rows_a, _ACC[0].at[d_cur.at[c0]],
                                          ssem_a).wait()
                    pltpu.async_copy(tab.at[c].at[s_cur.at[c0]], rows_a,
                                     gsem_a)
                    pltpu.make_async_copy(rows_b, _ACC[0].at[d_cur.at[c1]],
                                          ssem_b).wait()
                    pltpu.async_copy(tab.at[c].at[s_cur.at[c1]], rows_b,
                                     gsem_b)
                    pltpu.make_async_copy(tab.at[c].at[s_cur.at[c0]], rows_a,
                                          gsem_a).wait()
                    pltpu.async_copy(rows_a, _ACC[0].at[d_cur.at[c0]], ssem_a,
                                     add=True)
                    pltpu.make_async_copy(tab.at[c].at[s_cur.at[c1]], rows_b,
                                          gsem_b).wait()
                    pltpu.async_copy(rows_b, _ACC[0].at[d_cur.at[c1]], ssem_b,
                                     add=True)

                pltpu.make_async_copy(rows_a, _ACC[0].at[d_cur.at[_IG - 2]],
                                      ssem_a).wait()
                pltpu.make_async_copy(rows_b, _ACC[0].at[d_cur.at[_IG - 1]],
                                      ssem_b).wait()
                if g + 1 < _NG:
                    pltpu.make_async_copy(
                        src_ref.at[r, s, pl.ds((g + 1) * _IG, _IG)], s_nxt,
                        isem).wait()
                    pltpu.make_async_copy(
                        dst_ref.at[r, s, pl.ds((g + 1) * _IG, _IG)], d_nxt,
                        isem).wait()

            plsc.subcore_barrier()
            pltpu.sync_copy(_ACC[0].at[pl.ds(row0, _RPT)],
                            out_ref.at[r, c, pl.ds(row0, _RPT)])

    _ACC = [None]

    def body_wrap(h_ref, src_ref, dst_ref, z_ref, out_ref,
                  sa_v, sb_v, da_v, db_v, rows_a, rows_b, acc_sh,
                  isem, gsem_a, gsem_b, ssem_a, ssem_b):
        _ACC[0] = acc_sh
        body(h_ref, src_ref, dst_ref, z_ref, out_ref,
             sa_v, sb_v, da_v, db_v, rows_a, rows_b,
             isem, gsem_a, gsem_b, ssem_a, ssem_b)

    k = pl.kernel(
        body_wrap,
        out_type=jax.ShapeDtypeStruct((4, _NC, _NP, _HH), jnp.float32),
        mesh=mesh,
        scratch_types=[
            pltpu.VMEM((_IG, _CH), jnp.int32),
            pltpu.VMEM((_IG, _CH), jnp.int32),
            pltpu.VMEM((_IG, _CH), jnp.int32),
            pltpu.VMEM((_IG, _CH), jnp.int32),
            pltpu.VMEM((_CH, _HH), jnp.float32),
            pltpu.VMEM((_CH, _HH), jnp.float32),
            pltpu.VMEM_SHARED((_NP, _HH), jnp.float32),
            pltpu.SemaphoreType.DMA,
            pltpu.SemaphoreType.DMA,
            pltpu.SemaphoreType.DMA,
            pltpu.SemaphoreType.DMA,
            pltpu.SemaphoreType.DMA,
        ],
        compiler_params=pltpu.CompilerParams(
            needs_layout_passes=False, use_tc_tiling_on_sc=False),
        name="sc_segsum",
    )
    return k(h_stack, src_all, dst_all, zeros_hbm)


def _sc_counts(dst_all):
    mesh = plsc.VectorSubcoreMesh(core_axis_name="c", subcore_axis_name="s")

    def body(dst_ref, cnt_ref, dst_v, hist_v, sem):
        c = lax.axis_index("c")
        s = lax.axis_index("s")
        zeros16 = jnp.zeros((16,), jnp.float32)
        ones16 = jnp.ones((16,), jnp.float32)
        for r in range(4):
            @pl.when(c == r // 2)
            def _rel():
                @pl.loop(0, _NP // 16)
                def _hz(i):
                    hist_v[pl.ds(i * 16, 16)] = zeros16

                for g in range(_NG):
                    pltpu.sync_copy(dst_ref.at[r, s, pl.ds(g * _IG, _IG)],
                                    dst_v)

                    @pl.loop(0, _IG)
                    def _chunk(j):
                        for gg in range(_CH // 16):
                            idx = dst_v[j, pl.ds(gg * 16, 16)]
                            plsc.addupdate_scatter(hist_v, [idx], ones16)

                pltpu.sync_copy(hist_v,
                                cnt_ref.at[pl.ds((s * 4 + r) * _NP, _NP)])

    return pl.kernel(
        body,
        out_type=jax.ShapeDtypeStruct((_NS * 4 * _NP,), jnp.float32),
        mesh=mesh,
        scratch_types=[
            pltpu.VMEM((_IG, _CH), jnp.int32),
            pltpu.VMEM((_NP,), jnp.float32),
            pltpu.SemaphoreType.DMA,
        ],
        compiler_params=pltpu.CompilerParams(
            needs_layout_passes=False, use_tc_tiling_on_sc=False),
        name="sc_counts",
    )(dst_all)


def _tc_project(xs, Wps, bps):
    def body(x_ref, w_ref, b_ref, o_ref):
        o_ref[0, 0] = (
            jnp.dot(x_ref[0], w_ref[0], preferred_element_type=jnp.float32)
            + b_ref[0, 0]
        )

    return pl.pallas_call(
        body,
        grid=(3, _NB, _NC),
        in_specs=[
            pl.BlockSpec((1, _BN, _D), lambda t, i, c: (t, i, 0)),
            pl.BlockSpec((1, _D, _HH), lambda t, i, c: (t, 0, c)),
            pl.BlockSpec((1, 1, 1, _HH), lambda t, i, c: (t, c, 0, 0)),
        ],
        out_specs=pl.BlockSpec((1, 1, _BN, _HH), lambda t, i, c: (t, c, i, 0)),
        out_shape=jax.ShapeDtypeStruct((3, _NC, _NP, _HH), jnp.float32),
        name="tc_project",
    )(xs, Wps, bps)


def _tc_layer_combine(sums, cnt, h_stack, Wl_l, bl_l, Wr_l):
    def body(s_ref, c_ref, h_ref, wl_ref, bl_ref, wr_ref, o_ref):
        cnts = jnp.sum(c_ref[...], axis=0)
        o = []
        for r in range(4):
            acc = jnp.dot(s_ref[r, 0], wl_ref[r, :_HH, :],
                          preferred_element_type=jnp.float32)
            acc = acc + jnp.dot(s_ref[r, 1], wl_ref[r, _HH:, :],
                                preferred_element_type=jnp.float32)
            inv = 1.0 / jnp.maximum(cnts[r][:, None], 1.0)
            acc = acc * inv + bl_ref[r]
            t = _DST_T[r]
            acc = acc + jnp.dot(h_ref[t, 0], wr_ref[r, :_HH, :],
                                preferred_element_type=jnp.float32)
            acc = acc + jnp.dot(h_ref[t, 1], wr_ref[r, _HH:, :],
                                preferred_element_type=jnp.float32)
            o.append(acc)
        new = (jnp.maximum(o[1], 0.0),
               jnp.maximum((o[0] + o[3]) * 0.5, 0.0),
               jnp.maximum(o[2], 0.0))
        for t in range(3):
            o_ref[t, 0] = new[t][:, :_HH]
            o_ref[t, 1] = new[t][:, _HH:]

    return pl.pallas_call(
        body,
        grid=(_NB,),
        in_specs=[
            pl.BlockSpec((4, _NC, _BN, _HH), lambda i: (0, 0, i, 0)),
            pl.BlockSpec((_NS, 4, _BN), lambda i: (0, 0, i)),
            pl.BlockSpec((3, _NC, _BN, _HH), lambda i: (0, 0, i, 0)),
            pl.BlockSpec((4, _H, _H), lambda i: (0, 0, 0)),
            pl.BlockSpec((4, _H), lambda i: (0, 0)),
            pl.BlockSpec((4, _H, _H), lambda i: (0, 0, 0)),
        ],
        out_specs=pl.BlockSpec((3, _NC, _BN, _HH), lambda i: (0, 0, i, 0)),
        out_shape=jax.ShapeDtypeStruct((3, _NC, _NP, _HH), jnp.float32),
        name="tc_layer_combine",
    )(sums, cnt, h_stack, Wl_l, bl_l, Wr_l)


def _tc_classifier(h_stack, Wc, bc):
    def body(h_ref, w_ref, b_ref, o_ref):
        o_ref[...] = (
            jnp.dot(h_ref[0, 0], w_ref[:_HH, :],
                    preferred_element_type=jnp.float32)
            + jnp.dot(h_ref[0, 1], w_ref[_HH:, :],
                      preferred_element_type=jnp.float32)
            + b_ref[...]
        )

    return pl.pallas_call(
        body,
        grid=(_NB,),
        in_specs=[
            pl.BlockSpec((1, _NC, _BN, _HH), lambda i: (0, 0, i, 0)),
            pl.BlockSpec((_H, _C), lambda i: (0, 0)),
            pl.BlockSpec((1, _C), lambda i: (0, 0)),
        ],
        out_specs=pl.BlockSpec((_BN, _C), lambda i: (i, 0)),
        out_shape=jax.ShapeDtypeStruct((_NP, _C), jnp.float32),
        name="tc_classifier",
    )(h_stack, Wc, bc.reshape(1, _C))


def kernel(x_lncRNA, x_miRNA, x_mRNA, edge_index_interacts,
           edge_index_rev_interacts, edge_index_regulates,
           edge_index_rev_regulates, Wp_lncRNA, bp_lncRNA, Wp_miRNA, bp_miRNA,
           Wp_mRNA, bp_mRNA, Wl, bl, Wr, Wc, bc):
    xs = jnp.stack([x_lncRNA, x_miRNA, x_mRNA])
    xs = jnp.pad(xs, ((0, 0), (0, _NP - _N), (0, 0)))
    Wps = jnp.stack([Wp_lncRNA, Wp_miRNA, Wp_mRNA])
    bps = jnp.stack([bp_lncRNA, bp_miRNA, bp_mRNA]).reshape(3, _NC, 1, _HH)

    eis = (edge_index_interacts, edge_index_rev_interacts,
           edge_index_regulates, edge_index_rev_regulates)
    pad_row = jnp.int32(_NP - 1)

    def _prep(row):
        row = row.reshape(_NS, _EPT)
        row = jnp.pad(row, ((0, 0), (0, _EPP - _EPT)), constant_values=pad_row)
        return row.reshape(_NS, _NCHUNK, _CH)

    src_all = jnp.stack([_prep(e[0]) for e in eis])
    dst_all = jnp.stack([_prep(e[1]) for e in eis])

    h = _tc_project(xs, Wps, bps)

    cnt = _sc_counts(dst_all).reshape(_NS, 4, _NP)
    zeros_hbm = jnp.zeros((_RPT, _HH), jnp.float32)
    for l in range(_L):
        sums = _sc_segment_sums(h, src_all, dst_all, zeros_hbm)
        h = _tc_layer_combine(sums, cnt, h, Wl[l], bl[l], Wr[l])

    return _tc_classifier(h, Wc, bc)[:_N]


# speedup vs baseline: 2.4489x; 1.4217x over previous
import functools
import jax
import jax.numpy as jnp
from jax import lax
from jax.experimental import pallas as pl
from jax.experimental.pallas import tpu as pltpu
from jax.experimental.pallas import tpu_sc as plsc

_N = 10000
_D = 256
_H = 256
_C = 64
_E = 160000
_L = 2

_NC = 2
_NS = 16
_HH = _H // 2
_EPT = _E // _NS
_CH = 128
_EPP = 10240
_NCHUNK = _EPP // _CH
_NP = 10240
_RPT = _NP // _NS
_ZCH = 64
_NZ = _RPT // _ZCH
_IG = 16
_NG = _NCHUNK // _IG

_SRC_T = (0, 1, 1, 2)
_DST_T = (1, 0, 2, 1)

_BN = 1024
_NB = _NP // _BN


def _sc_segment_sums(h_stack, src_all, dst_all, zeros_hbm):
    mesh = plsc.VectorSubcoreMesh(core_axis_name="c", subcore_axis_name="s")

    def body(h_ref, src_ref, dst_ref, z_ref, out_ref,
             sa_v, sb_v, da_v, db_v, rows_a, rows_b,
             isem, gsem_a, gsem_b, ssem_a, ssem_b):
        c = lax.axis_index("c")
        s = lax.axis_index("s")
        row0 = s * _RPT

        for r in range(4):
            tab = h_ref.at[_SRC_T[r]]
            pltpu.sync_copy(z_ref, _ACC[0].at[pl.ds(row0, _RPT)])
            plsc.subcore_barrier()

            bufs = ((sa_v, da_v, rows_a), (sb_v, db_v, rows_b))
            pltpu.sync_copy(src_ref.at[r, s, pl.ds(0, _IG)], sa_v)
            pltpu.sync_copy(dst_ref.at[r, s, pl.ds(0, _IG)], da_v)
            for g in range(_NG):
                s_cur, d_cur, _ = bufs[g % 2]
                s_nxt, d_nxt, _ = bufs[(g + 1) % 2]
                if g + 1 < _NG:
                    pltpu.async_copy(
                        src_ref.at[r, s, pl.ds((g + 1) * _IG, _IG)], s_nxt,
                        isem)
                    pltpu.async_copy(
                        dst_ref.at[r, s, pl.ds((g + 1) * _IG, _IG)], d_nxt,
                        isem)

                pltpu.async_copy(tab.at[c].at[s_cur.at[0]], rows_a, gsem_a)
                pltpu.async_copy(tab.at[c].at[s_cur.at[1]], rows_b, gsem_b)
                pltpu.make_async_copy(tab.at[c].at[s_cur.at[0]], rows_a,
                                      gsem_a).wait()
                pltpu.async_copy(rows_a, _ACC[0].at[d_cur.at[0]], ssem_a,
                                 add=True)
                pltpu.make_async_copy(tab.at[c].at[s_cur.at[1]], rows_b,
                                      gsem_b).wait()
                pltpu.async_copy(rows_b, _ACC[0].at[d_cur.at[1]], ssem_b,
                                 add=True)

                @pl.loop(1, _IG // 2)
                def _pair(p):
                    c0 = 2 * p
                    c1 = 2 * p + 1
                    pltpu.make_async_copy(rows_a, _ACC[0].at[d_cur.at[c0]],
                                          ssem_a).wait()
                    pltpu.async_copy(tab.at[c].at[s_cur.at[c0]], rows_a,
                                     gsem_a)
                    pltpu.make_async_copy(rows_b, _ACC[0].at[d_cur.at[c1]],
                                          ssem_b).wait()
                    pltpu.async_copy(tab.at[c].at[s_cur.at[c1]], rows_b,
                                     gsem_b)
                    pltpu.make_async_copy(tab.at[c].at[s_cur.at[c0]], rows_a,
                                          gsem_a).wait()
                    pltpu.async_copy(rows_a, _ACC[0].at[d_cur.at[c0]], ssem_a,
                                     add=True)
                    pltpu.make_async_copy(tab.at[c].at[s_cur.at[c1]], rows_b,
                                          gsem_b).wait()
                    pltpu.async_copy(rows_b, _ACC[0].at[d_cur.at[c1]], ssem_b,
                                     add=True)

                pltpu.make_async_copy(rows_a, _ACC[0].at[d_cur.at[_IG - 2]],
                                      ssem_a).wait()
                pltpu.make_async_copy(rows_b, _ACC[0].at[d_cur.at[_IG - 1]],
                                      ssem_b).wait()
                if g + 1 < _NG:
                    pltpu.make_async_copy(
                        src_ref.at[r, s, pl.ds((g + 1) * _IG, _IG)], s_nxt,
                        isem).wait()
                    pltpu.make_async_copy(
                        dst_ref.at[r, s, pl.ds((g + 1) * _IG, _IG)], d_nxt,
                        isem).wait()

            plsc.subcore_barrier()
            pltpu.sync_copy(_ACC[0].at[pl.ds(row0, _RPT)],
                            out_ref.at[r, c, pl.ds(row0, _RPT)])

    _ACC = [None]

    def body_wrap(h_ref, src_ref, dst_ref, z_ref, out_ref,
                  sa_v, sb_v, da_v, db_v, rows_a, rows_b, acc_sh,
                  isem, gsem_a, gsem_b, ssem_a, ssem_b):
        _ACC[0] = acc_sh
        body(h_ref, src_ref, dst_ref, z_ref, out_ref,
             sa_v, sb_v, da_v, db_v, rows_a, rows_b,
             isem, gsem_a, gsem_b, ssem_a, ssem_b)

    k = pl.kernel(
        body_wrap,
        out_type=jax.ShapeDtypeStruct((4, _NC, _NP, _HH), jnp.bfloat16),
        mesh=mesh,
        scratch_types=[
            pltpu.VMEM((_IG, _CH), jnp.int32),
            pltpu.VMEM((_IG, _CH), jnp.int32),
            pltpu.VMEM((_IG, _CH), jnp.int32),
            pltpu.VMEM((_IG, _CH), jnp.int32),
            pltpu.VMEM((_CH, _HH), jnp.bfloat16),
            pltpu.VMEM((_CH, _HH), jnp.bfloat16),
            pltpu.VMEM_SHARED((_NP, _HH), jnp.bfloat16),
            pltpu.SemaphoreType.DMA,
            pltpu.SemaphoreType.DMA,
            pltpu.SemaphoreType.DMA,
            pltpu.SemaphoreType.DMA,
            pltpu.SemaphoreType.DMA,
        ],
        compiler_params=pltpu.CompilerParams(
            needs_layout_passes=False, use_tc_tiling_on_sc=False),
        name="sc_segsum",
    )
    return k(h_stack, src_all, dst_all, zeros_hbm)


def _sc_counts(dst_all):
    mesh = plsc.VectorSubcoreMesh(core_axis_name="c", subcore_axis_name="s")

    def body(dst_ref, cnt_ref, dst_v, hist_v, sem):
        c = lax.axis_index("c")
        s = lax.axis_index("s")
        zeros16 = jnp.zeros((16,), jnp.float32)
        ones16 = jnp.ones((16,), jnp.float32)
        for r in range(4):
            @pl.when(c == r // 2)
            def _rel():
                @pl.loop(0, _NP // 16)
                def _hz(i):
                    hist_v[pl.ds(i * 16, 16)] = zeros16

                for g in range(_NG):
                    pltpu.sync_copy(dst_ref.at[r, s, pl.ds(g * _IG, _IG)],
                                    dst_v)

                    @pl.loop(0, _IG)
                    def _chunk(j):
                        for gg in range(_CH // 16):
                            idx = dst_v[j, pl.ds(gg * 16, 16)]
                            plsc.addupdate_scatter(hist_v, [idx], ones16)

                pltpu.sync_copy(hist_v,
                                cnt_ref.at[pl.ds((s * 4 + r) * _NP, _NP)])

    return pl.kernel(
        body,
        out_type=jax.ShapeDtypeStruct((_NS * 4 * _NP,), jnp.float32),
        mesh=mesh,
        scratch_types=[
            pltpu.VMEM((_IG, _CH), jnp.int32),
            pltpu.VMEM((_NP,), jnp.float32),
            pltpu.SemaphoreType.DMA,
        ],
        compiler_params=pltpu.CompilerParams(
            needs_layout_passes=False, use_tc_tiling_on_sc=False),
        name="sc_counts",
    )(dst_all)


def _tc_project(xs, Wps, bps):
    def body(x_ref, w_ref, b_ref, o_ref):
        o_ref[0, 0] = (
            jnp.dot(x_ref[0], w_ref[0], preferred_element_type=jnp.float32)
            + b_ref[0, 0]
        ).astype(jnp.bfloat16)

    return pl.pallas_call(
        body,
        grid=(3, _NB, _NC),
        in_specs=[
            pl.BlockSpec((1, _BN, _D), lambda t, i, c: (t, i, 0)),
            pl.BlockSpec((1, _D, _HH), lambda t, i, c: (t, 0, c)),
            pl.BlockSpec((1, 1, 1, _HH), lambda t, i, c: (t, c, 0, 0)),
        ],
        out_specs=pl.BlockSpec((1, 1, _BN, _HH), lambda t, i, c: (t, c, i, 0)),
        out_shape=jax.ShapeDtypeStruct((3, _NC, _NP, _HH), jnp.bfloat16),
        name="tc_project",
    )(xs, Wps, bps)


def _tc_layer_combine(sums, cnt, h_stack, Wl_l, bl_l, Wr_l):
    def body(s_ref, c_ref, h_ref, wl_ref, bl_ref, wr_ref, o_ref):
        cnts = jnp.sum(c_ref[...], axis=0)
        o = []
        for r in range(4):
            acc = jnp.dot(s_ref[r, 0].astype(jnp.float32), wl_ref[r, :_HH, :],
                          preferred_element_type=jnp.float32)
            acc = acc + jnp.dot(s_ref[r, 1].astype(jnp.float32),
                                wl_ref[r, _HH:, :],
                                preferred_element_type=jnp.float32)
            inv = 1.0 / jnp.maximum(cnts[r][:, None], 1.0)
            acc = acc * inv + bl_ref[r]
            t = _DST_T[r]
            acc = acc + jnp.dot(h_ref[t, 0].astype(jnp.float32),
                                wr_ref[r, :_HH, :],
                                preferred_element_type=jnp.float32)
            acc = acc + jnp.dot(h_ref[t, 1].astype(jnp.float32),
                                wr_ref[r, _HH:, :],
                                preferred_element_type=jnp.float32)
            o.append(acc)
        new = (jnp.maximum(o[1], 0.0),
               jnp.maximum((o[0] + o[3]) * 0.5, 0.0),
               jnp.maximum(o[2], 0.0))
        for t in range(3):
            o_ref[t, 0] = new[t][:, :_HH].astype(jnp.bfloat16)
            o_ref[t, 1] = new[t][:, _HH:].astype(jnp.bfloat16)

    return pl.pallas_call(
        body,
        grid=(_NB,),
        in_specs=[
            pl.BlockSpec((4, _NC, _BN, _HH), lambda i: (0, 0, i, 0)),
            pl.BlockSpec((_NS, 4, _BN), lambda i: (0, 0, i)),
            pl.BlockSpec((3, _NC, _BN, _HH), lambda i: (0, 0, i, 0)),
            pl.BlockSpec((4, _H, _H), lambda i: (0, 0, 0)),
            pl.BlockSpec((4, _H), lambda i: (0, 0)),
            pl.BlockSpec((4, _H, _H), lambda i: (0, 0, 0)),
        ],
        out_specs=pl.BlockSpec((3, _NC, _BN, _HH), lambda i: (0, 0, i, 0)),
        out_shape=jax.ShapeDtypeStruct((3, _NC, _NP, _HH), jnp.bfloat16),
        name="tc_layer_combine",
    )(sums, cnt, h_stack, Wl_l, bl_l, Wr_l)


def _tc_classifier(h_stack, Wc, bc):
    def body(h_ref, w_ref, b_ref, o_ref):
        o_ref[...] = (
            jnp.dot(h_ref[0, 0].astype(jnp.float32), w_ref[:_HH, :],
                    preferred_element_type=jnp.float32)
            + jnp.dot(h_ref[0, 1].astype(jnp.float32), w_ref[_HH:, :],
                      preferred_element_type=jnp.float32)
            + b_ref[...]
        )

    return pl.pallas_call(
        body,
        grid=(_NB,),
        in_specs=[
            pl.BlockSpec((1, _NC, _BN, _HH), lambda i: (0, 0, i, 0)),
            pl.BlockSpec((_H, _C), lambda i: (0, 0)),
            pl.BlockSpec((1, _C), lambda i: (0, 0)),
        ],
        out_specs=pl.BlockSpec((_BN, _C), lambda i: (i, 0)),
        out_shape=jax.ShapeDtypeStruct((_NP, _C), jnp.float32),
        name="tc_classifier",
    )(h_stack, Wc, bc.reshape(1, _C))


def kernel(x_lncRNA, x_miRNA, x_mRNA, edge_index_interacts,
           edge_index_rev_interacts, edge_index_regulates,
           edge_index_rev_regulates, Wp_lncRNA, bp_lncRNA, Wp_miRNA, bp_miRNA,
           Wp_mRNA, bp_mRNA, Wl, bl, Wr, Wc, bc):
    xs = jnp.stack([x_lncRNA, x_miRNA, x_mRNA])
    xs = jnp.pad(xs, ((0, 0), (0, _NP - _N), (0, 0)))
    Wps = jnp.stack([Wp_lncRNA, Wp_miRNA, Wp_mRNA])
    bps = jnp.stack([bp_lncRNA, bp_miRNA, bp_mRNA]).reshape(3, _NC, 1, _HH)

    eis = (edge_index_interacts, edge_index_rev_interacts,
           edge_index_regulates, edge_index_rev_regulates)
    pad_row = jnp.int32(_NP - 1)

    def _prep(row):
        row = row.reshape(_NS, _EPT)
        row = jnp.pad(row, ((0, 0), (0, _EPP - _EPT)), constant_values=pad_row)
        return row.reshape(_NS, _NCHUNK, _CH)

    src_all = jnp.stack([_prep(e[0]) for e in eis])
    dst_all = jnp.stack([_prep(e[1]) for e in eis])

    h = _tc_project(xs, Wps, bps)

    cnt = _sc_counts(dst_all).reshape(_NS, 4, _NP)
    zeros_hbm = jnp.zeros((_RPT, _HH), jnp.bfloat16)
    for l in range(_L):
        sums = _sc_segment_sums(h, src_all, dst_all, zeros_hbm)
        h = _tc_layer_combine(sums, cnt, h, Wl[l], bl[l], Wr[l])

    return _tc_classifier(h, Wc, bc)[:_N]
